# baseline (device time: 13464 ns/iter reference)
import jax
import jax.numpy as jnp
from jax import lax
from jax.experimental import pallas as pl
from jax.experimental.pallas import tpu as pltpu

N_DEV = 4
N_TOK = 256
D_IN = 128
D_OUT = 256
N_EXP = 8
CAP = 25
E_PER = N_EXP // N_DEV
CHUNK = N_TOK // N_DEV


def kernel(x, router_W, route_idx, expert_W):
    del router_W

    def body(x_ref, idx_ref, w_ref, out_ref,
             partial_ref, comm_ref, send_sems, recv_sems):
        my = lax.axis_index("i")
        left = (my - 1) % N_DEV
        right = (my + 1) % N_DEV

        barrier_sem = pltpu.get_barrier_semaphore()
        for nbr in (left, right):
            pl.semaphore_signal(barrier_sem, inc=1, device_id=(nbr,),
                                device_id_type=pl.DeviceIdType.MESH)
        pl.semaphore_wait(barrier_sem, 2)

        e_col = idx_ref[:, :]
        exp_row = lax.broadcasted_iota(jnp.int32, (N_TOK, N_EXP), 1)
        onehot = (exp_row == e_col).astype(jnp.float32)
        row_i = lax.broadcasted_iota(jnp.int32, (N_TOK, N_TOK), 0)
        col_j = lax.broadcasted_iota(jnp.int32, (N_TOK, N_TOK), 1)
        tri = (col_j < row_i).astype(jnp.float32)
        ranks = jnp.dot(tri, onehot, preferred_element_type=jnp.float32)
        rank = jnp.sum(ranks * onehot, axis=1, keepdims=True)
        keep = rank < float(CAP)

        acc = jnp.zeros((N_TOK, D_OUT), jnp.float32)
        for l in range(E_PER):
            ge = my * E_PER + l
            m = jnp.where((e_col == ge) & keep, 1.0, 0.0)
            xm = x_ref[:, :] * m
            acc = acc + jnp.dot(xm, w_ref[l], preferred_element_type=jnp.float32)
        partial_ref[:, :] = acc

        first = (my + N_DEV - 1) % N_DEV
        comm_ref[0, :, :] = partial_ref[pl.ds(first * CHUNK, CHUNK), :]

        for t in range(1, N_DEV):
            s, r = t - 1, t
            rdma = pltpu.make_async_remote_copy(
                src_ref=comm_ref.at[s],
                dst_ref=comm_ref.at[r],
                send_sem=send_sems.at[s],
                recv_sem=recv_sems.at[s],
                device_id=(right,),
                device_id_type=pl.DeviceIdType.MESH,
            )
            rdma.start()
            rdma.wait()
            c = (my + 2 * N_DEV - 1 - t) % N_DEV
            comm_ref[r, :, :] = (
                comm_ref[r, :, :] + partial_ref[pl.ds(c * CHUNK, CHUNK), :]
            )

        out_ref[:, :] = comm_ref[N_DEV - 1, :, :]

    return pl.pallas_call(
        body,
        out_shape=jax.ShapeDtypeStruct((CHUNK, D_OUT), jnp.float32),
        in_specs=[
            pl.BlockSpec(memory_space=pltpu.VMEM),
            pl.BlockSpec(memory_space=pltpu.VMEM),
            pl.BlockSpec(memory_space=pltpu.VMEM),
        ],
        out_specs=pl.BlockSpec(memory_space=pltpu.VMEM),
        scratch_shapes=[
            pltpu.VMEM((N_TOK, D_OUT), jnp.float32),
            pltpu.VMEM((N_DEV, CHUNK, D_OUT), jnp.float32),
            pltpu.SemaphoreType.DMA((N_DEV - 1,)),
            pltpu.SemaphoreType.DMA((N_DEV - 1,)),
        ],
        compiler_params=pltpu.CompilerParams(collective_id=0),
    )(x, route_idx, expert_W)


# device time: 8762 ns/iter; 1.5366x vs baseline; 1.5366x over previous
import jax
import jax.numpy as jnp
from jax import lax
from jax.experimental import pallas as pl
from jax.experimental.pallas import tpu as pltpu

N_DEV = 4
N_TOK = 256
D_IN = 128
D_OUT = 256
N_EXP = 8
CAP = 25
E_PER = N_EXP // N_DEV
CHUNK = N_TOK // N_DEV


def kernel(x, router_W, route_idx, expert_W):
    del router_W

    def body(x_ref, idx_ref, w_ref, out_ref,
             partial_ref, recv_ref, send_sems, recv_sems):
        my = lax.axis_index("i")

        barrier_sem = pltpu.get_barrier_semaphore()
        for d in range(1, N_DEV):
            pl.semaphore_signal(barrier_sem, inc=1,
                                device_id=((my + d) % N_DEV,),
                                device_id_type=pl.DeviceIdType.MESH)
        pl.semaphore_wait(barrier_sem, N_DEV - 1)

        e_col = idx_ref[:, :]
        exp_row = lax.broadcasted_iota(jnp.int32, (N_TOK, N_EXP), 1)
        onehot = (exp_row == e_col).astype(jnp.float32)
        row_i = lax.broadcasted_iota(jnp.int32, (N_TOK, N_TOK), 0)
        col_j = lax.broadcasted_iota(jnp.int32, (N_TOK, N_TOK), 1)
        tri = (col_j < row_i).astype(jnp.float32)
        ranks = jnp.dot(tri, onehot, preferred_element_type=jnp.float32)
        rank = jnp.sum(ranks * onehot, axis=1, keepdims=True)
        keep = rank < float(CAP)

        acc = jnp.zeros((N_TOK, D_OUT), jnp.float32)
        for l in range(E_PER):
            ge = my * E_PER + l
            m = jnp.where((e_col == ge) & keep, 1.0, 0.0)
            xm = x_ref[:, :] * m
            acc = acc + jnp.dot(xm, w_ref[l], preferred_element_type=jnp.float32)
        partial_ref[:, :] = acc

        rdmas = []
        for d in range(1, N_DEV):
            q = (my + d) % N_DEV
            rdma = pltpu.make_async_remote_copy(
                src_ref=partial_ref.at[pl.ds(q * CHUNK, CHUNK), :],
                dst_ref=recv_ref.at[d - 1],
                send_sem=send_sems.at[d - 1],
                recv_sem=recv_sems.at[d - 1],
                device_id=(q,),
                device_id_type=pl.DeviceIdType.MESH,
            )
            rdma.start()
            rdmas.append(rdma)

        for rdma in rdmas:
            rdma.wait_recv()
        out_ref[:, :] = (
            partial_ref[pl.ds(my * CHUNK, CHUNK), :]
            + recv_ref[0, :, :] + recv_ref[1, :, :] + recv_ref[2, :, :]
        )
        for rdma in rdmas:
            rdma.wait_send()

    return pl.pallas_call(
        body,
        out_shape=jax.ShapeDtypeStruct((CHUNK, D_OUT), jnp.float32),
        in_specs=[
            pl.BlockSpec(memory_space=pltpu.VMEM),
            pl.BlockSpec(memory_space=pltpu.VMEM),
            pl.BlockSpec(memory_space=pltpu.VMEM),
        ],
        out_specs=pl.BlockSpec(memory_space=pltpu.VMEM),
        scratch_shapes=[
            pltpu.VMEM((N_TOK, D_OUT), jnp.float32),
            pltpu.VMEM((N_DEV - 1, CHUNK, D_OUT), jnp.float32),
            pltpu.SemaphoreType.DMA((N_DEV - 1,)),
            pltpu.SemaphoreType.DMA((N_DEV - 1,)),
        ],
        compiler_params=pltpu.CompilerParams(collective_id=0),
    )(x, route_idx, expert_W)
